# baseline (speedup 1.0000x reference)
import functools

import jax
import jax.numpy as jnp
from jax import lax
from jax.experimental import pallas as pl
from jax.experimental.pallas import tpu as pltpu
from jax.experimental.pallas import tpu_sc as plsc

_L = 16


@functools.lru_cache(maxsize=None)
def _build(B, D):
    info = plsc.get_sparse_core_info()
    NC, NS = info.num_cores, info.num_subcores
    NW = NC * NS
    assert B % NW == 0
    BPW = B // NW
    CH = 128
    NCH = BPW // CH
    assert BPW % CH == 0 and BPW % _L == 0

    mesh = plsc.VectorSubcoreMesh(core_axis_name="c", subcore_axis_name="s")
    cparams = pltpu.CompilerParams(
        needs_layout_passes=False, use_tc_tiling_on_sc=False
    )

    @functools.partial(
        pl.kernel,
        out_type=jax.ShapeDtypeStruct((B,), jnp.float32),
        mesh=mesh,
        compiler_params=cparams,
        scratch_types=[
            pltpu.VMEM((BPW,), jnp.int32),
            pltpu.VMEM((BPW,), jnp.int32),
            pltpu.VMEM((BPW, D), jnp.float32),
            pltpu.VMEM((BPW, D), jnp.float32),
            pltpu.VMEM((BPW, 1), jnp.float32),
            pltpu.VMEM((BPW, 1), jnp.float32),
            pltpu.VMEM((BPW,), jnp.float32),
            pltpu.VMEM((_L,), jnp.float32),
            pltpu.SemaphoreType.DMA,
        ],
    )
    def mf(uids_hbm, iids_hbm, utab_hbm, itab_hbm, ub_hbm, ib_hbm, gb_hbm,
           out_hbm, uid_v, iid_v, urows, irows, ubr, ibr, out_v, gb_v, sem):
        wid = lax.axis_index("s") * NC + lax.axis_index("c")
        base = wid * BPW
        pltpu.sync_copy(gb_hbm, gb_v)
        pltpu.sync_copy(uids_hbm.at[pl.ds(base, BPW)], uid_v)
        pltpu.sync_copy(iids_hbm.at[pl.ds(base, BPW)], iid_v)
        copies = []
        for k in range(NCH):
            sl = pl.ds(k * CH, CH)
            copies.append(pltpu.async_copy(utab_hbm.at[uid_v.at[sl]], urows.at[sl], sem))
            copies.append(pltpu.async_copy(itab_hbm.at[iid_v.at[sl]], irows.at[sl], sem))
            copies.append(pltpu.async_copy(ub_hbm.at[uid_v.at[sl]], ubr.at[sl], sem))
            copies.append(pltpu.async_copy(ib_hbm.at[iid_v.at[sl]], ibr.at[sl], sem))
        for c in copies:
            c.wait()

        gbv = gb_v[pl.ds(0, _L)]
        riota = lax.iota(jnp.int32, _L)
        zcol = jnp.zeros((_L,), jnp.int32)

        def group(g, carry):
            rbase = g * _L
            rows = rbase + riota
            acc = plsc.load_gather(ubr, [rows, zcol])
            acc = acc + plsc.load_gather(ibr, [rows, zcol]) + gbv
            for d in range(D):
                dv = jnp.full((_L,), d, jnp.int32)
                u = plsc.load_gather(urows, [rows, dv])
                v = plsc.load_gather(irows, [rows, dv])
                acc = acc + u * v
            out_v[pl.ds(rbase, _L)] = acc
            return carry

        lax.fori_loop(0, BPW // _L, group, 0)
        pltpu.sync_copy(out_v, out_hbm.at[pl.ds(base, BPW)])

    return mf


def kernel(user_ids, item_ids, user_table, item_table, user_bias, item_bias,
           global_bias):
    B = user_ids.shape[0]
    mf = _build(B, user_table.shape[1])
    return mf(user_ids.astype(jnp.int32), item_ids.astype(jnp.int32),
              user_table, item_table, user_bias, item_bias,
              jnp.broadcast_to(global_bias, (_L,)))


# speedup vs baseline: 2.4553x; 2.4553x over previous
import functools

import jax
import jax.numpy as jnp
from jax import lax
from jax.experimental import pallas as pl
from jax.experimental.pallas import tpu as pltpu
from jax.experimental.pallas import tpu_sc as plsc

_L = 16


@functools.lru_cache(maxsize=None)
def _build(B, D):
    info = plsc.get_sparse_core_info()
    NC, NS = info.num_cores, info.num_subcores
    NW = NC * NS
    assert B % NW == 0 and D % _L == 0
    BPW = B // NW
    CH = 128
    NCH = BPW // CH
    NDC = D // _L
    assert BPW % CH == 0 and BPW % _L == 0

    mesh = plsc.VectorSubcoreMesh(core_axis_name="c", subcore_axis_name="s")
    cparams = pltpu.CompilerParams(
        needs_layout_passes=False, use_tc_tiling_on_sc=False
    )
    @functools.partial(
        pl.kernel,
        out_type=jax.ShapeDtypeStruct((B,), jnp.float32),
        mesh=mesh,
        compiler_params=cparams,
        scratch_types=[
            pltpu.VMEM((BPW,), jnp.int32),
            pltpu.VMEM((BPW,), jnp.int32),
            pltpu.VMEM((BPW, D), jnp.float32),
            pltpu.VMEM((BPW, D), jnp.float32),
            pltpu.VMEM((BPW,), jnp.float32),
            pltpu.VMEM((BPW,), jnp.float32),
            pltpu.VMEM((BPW,), jnp.float32),
            pltpu.VMEM((_L,), jnp.float32),
            pltpu.SemaphoreType.DMA,
        ],
    )
    def mf(uids_hbm, iids_hbm, utab_hbm, itab_hbm, ub_hbm, ib_hbm, gb_hbm,
           out_hbm, uid_v, iid_v, urows, irows, ubr, ibr, out_v, gb_v, sem):
        wid = lax.axis_index("s") * NC + lax.axis_index("c")
        base = wid * BPW
        pltpu.sync_copy(gb_hbm, gb_v)
        pltpu.sync_copy(uids_hbm.at[pl.ds(base, BPW)], uid_v)
        pltpu.sync_copy(iids_hbm.at[pl.ds(base, BPW)], iid_v)
        copies = []
        for k in range(NCH):
            sl = pl.ds(k * CH, CH)
            copies.append(pltpu.async_copy(utab_hbm.at[uid_v.at[sl]], urows.at[sl], sem))
            copies.append(pltpu.async_copy(itab_hbm.at[iid_v.at[sl]], irows.at[sl], sem))
            copies.append(pltpu.async_copy(ub_hbm.at[uid_v.at[sl]], ubr.at[sl], sem))
            copies.append(pltpu.async_copy(ib_hbm.at[iid_v.at[sl]], ibr.at[sl], sem))
        for c in copies:
            c.wait()

        gbv = gb_v[pl.ds(0, _L)]
        lane = lax.iota(jnp.int32, _L)

        def group(g, carry):
            rbase = g * _L
            acc = ubr[pl.ds(rbase, _L)] + ibr[pl.ds(rbase, _L)] + gbv
            for j in range(_L):
                r = rbase + j
                s = urows[r, pl.ds(0, _L)] * irows[r, pl.ds(0, _L)]
                for cidx in range(1, NDC):
                    co = cidx * _L
                    s = s + urows[r, pl.ds(co, _L)] * irows[r, pl.ds(co, _L)]
                acc = acc + jnp.where(lane == j, jnp.sum(s), 0.0)
            out_v[pl.ds(rbase, _L)] = acc
            return carry

        lax.fori_loop(0, BPW // _L, group, 0)
        pltpu.sync_copy(out_v, out_hbm.at[pl.ds(base, BPW)])

    return mf


def kernel(user_ids, item_ids, user_table, item_table, user_bias, item_bias,
           global_bias):
    B = user_ids.shape[0]
    mf = _build(B, user_table.shape[1])
    return mf(user_ids.astype(jnp.int32), item_ids.astype(jnp.int32),
              user_table, item_table, user_bias.reshape(-1),
              item_bias.reshape(-1),
              jnp.broadcast_to(global_bias, (_L,)))
